# baseline (speedup 1.0000x reference)
import functools

import jax
import jax.numpy as jnp
from jax import lax
from jax.experimental import pallas as pl
from jax.experimental.pallas import tpu as pltpu
from jax.experimental.pallas import tpu_sc as plsc

ROWS = 131072
COLS = 14
NC = 2
NS = 16
L = 16
NW = NC * NS
CHUNK = ROWS * COLS // NW
GROUPS = ROWS // NW // L


def _tree_softmax_body(x_hbm, out_hbm, xin, xout):
    wid = lax.axis_index("s") * NC + lax.axis_index("c")
    base = wid * CHUNK
    pltpu.sync_copy(x_hbm.at[pl.ds(base, CHUNK)], xin)

    row_off = lax.iota(jnp.int32, L) * COLS

    def group(g, carry):
        i0 = row_off + g * (L * COLS)
        c = [plsc.load_gather(xin, [i0 + j]) for j in range(COLS)]
        s = [None] * COLS
        for k in range(COLS // 2):
            e = jnp.exp(c[2 * k + 1] - c[2 * k])
            sa = 1.0 / (1.0 + e)
            s[2 * k] = sa
            s[2 * k + 1] = 1.0 - sa
        o = [None] * COLS
        o[0], o[1] = s[0], s[1]
        for j in range(2, COLS):
            o[j] = s[j] * o[j // 2 - 1]
        for j in range(COLS):
            plsc.store_scatter(xout, [i0 + j], o[j])
        return carry

    lax.fori_loop(0, GROUPS, group, 0)
    pltpu.sync_copy(xout, out_hbm.at[pl.ds(base, CHUNK)])


@jax.jit
def kernel(input):
    mesh = plsc.VectorSubcoreMesh(core_axis_name="c", subcore_axis_name="s")
    flat = jnp.reshape(input, (ROWS * COLS,))
    run = pl.kernel(
        _tree_softmax_body,
        out_type=jax.ShapeDtypeStruct((ROWS * COLS,), jnp.float32),
        mesh=mesh,
        scratch_types=[
            pltpu.VMEM((CHUNK,), jnp.float32),
            pltpu.VMEM((CHUNK,), jnp.float32),
        ],
        compiler_params=pltpu.CompilerParams(needs_layout_passes=False),
    )
    return jnp.reshape(run(flat), (ROWS, COLS))
